# Initial kernel scaffold; baseline (speedup 1.0000x reference)
import jax
import jax.numpy as jnp
from jax.experimental import pallas as pl


def kernel(xyz):
    raise NotImplementedError("write your pallas kernel here")


# speedup vs baseline: 9.3134x; 9.3134x over previous
import functools

import jax
import jax.numpy as jnp
from jax import lax
from jax.experimental import pallas as pl
from jax.experimental.pallas import tpu as pltpu
from jax.experimental.pallas import tpu_sc as plsc

N = 8192
S = 512
L = 16
NC, NS = 2, 16
NW = NC * NS
CHUNKS = N // L


def _fps_body(x_hbm, y_hbm, z_hbm, ox_hbm, oy_hbm, oz_hbm,
              x_v, y_v, z_v, dist_v, ox_v, oy_v, oz_v):
    b = lax.axis_index("s") * NC + lax.axis_index("c")
    pltpu.sync_copy(x_hbm.at[b], x_v)
    pltpu.sync_copy(y_hbm.at[b], y_v)
    pltpu.sync_copy(z_hbm.at[b], z_v)

    big = jnp.full((L,), 1e10, jnp.float32)

    def init_body(i, carry):
        dist_v[pl.ds(i * L, L)] = big
        return carry

    lax.fori_loop(0, CHUNKS, init_body, 0)

    lane = lax.iota(jnp.int32, L)
    lane0 = lane == 0

    def outer(s, far):
        far_vec = jnp.broadcast_to(far, (L,))
        cx = plsc.load_gather(x_v, [far_vec])
        cy = plsc.load_gather(y_v, [far_vec])
        cz = plsc.load_gather(z_v, [far_vec])
        s_vec = jnp.broadcast_to(s, (L,))
        plsc.store_scatter(ox_v, [s_vec], cx, mask=lane0)
        plsc.store_scatter(oy_v, [s_vec], cy, mask=lane0)
        plsc.store_scatter(oz_v, [s_vec], cz, mask=lane0)

        def chunk(i, carry):
            bv, bi = carry
            sl = pl.ds(i * L, L)
            dx = x_v[sl] - cx
            dy = y_v[sl] - cy
            dz = z_v[sl] - cz
            d = dx * dx + dy * dy + dz * dz
            dd = jnp.minimum(dist_v[sl], d)
            dist_v[sl] = dd
            m = dd > bv
            bv = jnp.where(m, dd, bv)
            bi = jnp.where(m, i * L + lane, bi)
            return bv, bi

        bv0 = jnp.full((L,), -1.0, jnp.float32)
        bv, bi = lax.fori_loop(0, CHUNKS, chunk, (bv0, lane))

        m = jnp.max(bv)
        cand = jnp.where(bv == m, bi, jnp.int32(N))
        return jnp.min(cand)

    lax.fori_loop(0, S, outer, jnp.int32(0))

    pltpu.sync_copy(ox_v, ox_hbm.at[b])
    pltpu.sync_copy(oy_v, oy_hbm.at[b])
    pltpu.sync_copy(oz_v, oz_hbm.at[b])


@functools.lru_cache(maxsize=None)
def _build_fps_sc(interpret=False):
    return pl.kernel(
        _fps_body,
        out_type=(
            jax.ShapeDtypeStruct((NW, S), jnp.float32),
            jax.ShapeDtypeStruct((NW, S), jnp.float32),
            jax.ShapeDtypeStruct((NW, S), jnp.float32),
        ),
        mesh=plsc.VectorSubcoreMesh(core_axis_name="c", subcore_axis_name="s",
                                    num_cores=NC, num_subcores=NS),
        scratch_types=[
            pltpu.VMEM((N,), jnp.float32),
            pltpu.VMEM((N,), jnp.float32),
            pltpu.VMEM((N,), jnp.float32),
            pltpu.VMEM((N,), jnp.float32),
            pltpu.VMEM((S,), jnp.float32),
            pltpu.VMEM((S,), jnp.float32),
            pltpu.VMEM((S,), jnp.float32),
        ],
        compiler_params=pltpu.CompilerParams(needs_layout_passes=False),
        interpret=interpret,
    )


@jax.jit
def kernel(xyz):
    assert xyz.shape == (NW, N, 3)
    x = xyz[:, :, 0]
    y = xyz[:, :, 1]
    z = xyz[:, :, 2]
    ox, oy, oz = _build_fps_sc()(x, y, z)
    return jnp.stack([ox, oy, oz], axis=-1)
